# baseline (speedup 1.0000x reference)
import functools

import jax
import jax.numpy as jnp
from jax import lax
from jax.experimental import pallas as pl
from jax.experimental.pallas import tpu as pltpu
from jax.experimental.pallas import tpu_sc as plsc

_N = 10000
_E = 320000
_IN_C = 128
_HID_C = 64
_OUT_C = 40
_K = 10
_ALPHA = 0.1

_NT = 16
_NS = 640
_N_PAD = _NT * _NS
_C = 48
_BB = 128
_NB = 159
_EPT = _NB * _BB
_E_PAD = _NT * _EPT
_NSELF = 5
_NBE = _NB + _NSELF
_NCHUNK = _NS // _BB

_DEG_W = 16

_mesh = plsc.VectorSubcoreMesh(
    core_axis_name="c", subcore_axis_name="s", num_cores=1)
_sc_params = pltpu.CompilerParams(
    use_tc_tiling_on_sc=False, needs_layout_passes=False)


def _deg_body(dst_hbm, deg_out, dst_l, ones_b, deg_sh):
  tid = lax.axis_index("s")
  base = tid * _NS
  pltpu.sync_copy(dst_hbm.at[tid], dst_l)

  @pl.loop(0, _BB)
  def _(r):
    ones_b[r, pl.ds(0, 16)] = jnp.zeros((16,), jnp.float32)

  @pl.loop(0, _NCHUNK)
  def _(i):
    pltpu.sync_copy(ones_b, deg_sh.at[pl.ds(base + i * _BB, _BB)])

  @pl.loop(0, _BB)
  def _(r):
    ones_b[r, pl.ds(0, 16)] = jnp.ones((16,), jnp.float32)

  plsc.subcore_barrier()

  @pl.loop(0, _NB)
  def _(j):
    pltpu.sync_copy(ones_b, deg_sh.at[dst_l.at[j]], add=True)

  plsc.subcore_barrier()
  pltpu.sync_copy(deg_sh.at[pl.ds(base, _NS)], deg_out.at[pl.ds(base, _NS)])


_deg_kernel = functools.partial(
    pl.kernel,
    out_type=jax.ShapeDtypeStruct((_N_PAD, _DEG_W), jnp.float32),
    mesh=_mesh,
    compiler_params=_sc_params,
    scratch_types=[
        pltpu.VMEM((_NB, _BB), jnp.int32),
        pltpu.VMEM((_BB, _DEG_W), jnp.float32),
        pltpu.VMEM_SHARED((_N_PAD, _DEG_W), jnp.float32),
    ],
)(_deg_body)


def _prop_body(src_hbm, dst_hbm, v0_hbm, q_hbm, p_hbm, out_hbm,
               src_l, dst_l, q_l, B0, B1, vA, vB,
               gs0, gs1, ss0, ss1, ss2, ss3):
  tid = lax.axis_index("s")
  base = tid * _NS
  pltpu.sync_copy(src_hbm.at[tid], src_l)
  pltpu.sync_copy(dst_hbm.at[tid], dst_l)
  pltpu.sync_copy(q_hbm.at[pl.ds(base, _NS)], q_l)
  pltpu.sync_copy(v0_hbm.at[pl.ds(base, _NS)], vA.at[pl.ds(base, _NS)])

  def drain_g(buf, sem):
    pltpu.make_async_copy(vA.at[pl.ds(0, 2 * _BB)], buf, sem).wait()

  def drain_s(sem):
    pltpu.make_async_copy(vA.at[pl.ds(0, _BB)], B0.at[pl.ds(0, _BB)],
                          sem).wait()

  h0 = pl.ds(0, _BB)
  h1 = pl.ds(_BB, _BB)

  for t in range(_K):
    rd, wr = (vA, vB) if t % 2 == 0 else (vB, vA)

    @pl.loop(0, _BB)
    def _(r):
      for c in range(_C // 16):
        B0[r, pl.ds(c * 16, 16)] = jnp.zeros((16,), jnp.float32)

    @pl.loop(0, _NCHUNK)
    def _(i):
      pltpu.sync_copy(B0.at[h0], wr.at[pl.ds(base + i * _BB, _BB)])

    plsc.subcore_barrier()

    pltpu.async_copy(rd.at[src_l.at[0]], B0, gs0)
    pltpu.async_copy(rd.at[src_l.at[1]], B1, gs1)

    @pl.loop(0, _NBE // 4)
    def _(i):
      j = 4 * i
      drain_g(B0, gs0)
      pltpu.async_copy(B0.at[h0], wr.at[dst_l.at[j]], ss0, add=True)
      pltpu.async_copy(B0.at[h1], wr.at[dst_l.at[j + 1]], ss1, add=True)
      drain_g(B1, gs1)
      pltpu.async_copy(B1.at[h0], wr.at[dst_l.at[j + 2]], ss2, add=True)
      pltpu.async_copy(B1.at[h1], wr.at[dst_l.at[j + 3]], ss3, add=True)
      drain_s(ss0)
      drain_s(ss1)
      pltpu.async_copy(rd.at[src_l.at[2 * i + 2]], B0, gs0)
      drain_s(ss2)
      drain_s(ss3)
      pltpu.async_copy(rd.at[src_l.at[2 * i + 3]], B1, gs1)

    drain_g(B0, gs0)
    drain_g(B1, gs1)

    plsc.subcore_barrier()

    @pl.loop(0, _NCHUNK)
    def _(i):
      sl = pl.ds(base + i * _BB, _BB)
      d0 = pltpu.async_copy(wr.at[sl], B0.at[h0], gs0)
      d1 = pltpu.async_copy(p_hbm.at[sl], B0.at[h1], gs1)
      d0.wait()
      d1.wait()

      @pl.loop(0, _BB)
      def _(r):
        q16 = plsc.load_gather(q_l, [jnp.full((16,), i * _BB + r, jnp.int32)])
        for c in range(_C // 16):
          cs = pl.ds(c * 16, 16)
          B0[r, cs] = q16 * B0[r, cs] + B0[_BB + r, cs]

      pltpu.sync_copy(B0.at[h0], wr.at[sl])

  final = vA if _K % 2 == 0 else vB
  pltpu.sync_copy(final.at[pl.ds(base, _NS)], out_hbm.at[pl.ds(base, _NS)])


_prop_kernel = functools.partial(
    pl.kernel,
    out_type=jax.ShapeDtypeStruct((_N_PAD, _C), jnp.float32),
    mesh=_mesh,
    compiler_params=_sc_params,
    scratch_types=[
        pltpu.VMEM((_NBE // 2 + 2, 2 * _BB), jnp.int32),
        pltpu.VMEM((_NBE, _BB), jnp.int32),
        pltpu.VMEM((_NS,), jnp.float32),
        pltpu.VMEM((2 * _BB, _C), jnp.float32),
        pltpu.VMEM((2 * _BB, _C), jnp.float32),
        pltpu.VMEM_SHARED((_N_PAD, _C), jnp.float32),
        pltpu.VMEM_SHARED((_N_PAD, _C), jnp.float32),
        pltpu.SemaphoreType.DMA,
        pltpu.SemaphoreType.DMA,
        pltpu.SemaphoreType.DMA,
        pltpu.SemaphoreType.DMA,
        pltpu.SemaphoreType.DMA,
        pltpu.SemaphoreType.DMA,
    ],
)(_prop_body)


_RB = 1024


def _mlp_body(x_ref, w1_ref, b1_ref, w2_ref, b2_ref, deg_ref,
              v0_ref, q_ref, p_ref):
  h = jnp.maximum(jnp.dot(x_ref[...], w1_ref[...],
                          preferred_element_type=jnp.float32)
                  + b1_ref[...], 0.0)
  o = jnp.dot(h, w2_ref[...], preferred_element_type=jnp.float32) + b2_ref[...]
  o48 = jnp.concatenate([o, jnp.zeros((_RB, _C - _OUT_C), jnp.float32)],
                        axis=1)
  row = (pl.program_id(0) * _RB
         + lax.broadcasted_iota(jnp.int32, (_RB, 1), 0))
  deg = deg_ref[:, 0:1] + 1.0
  dinv = jnp.where(row < _N, lax.rsqrt(deg), 0.0)
  v0 = dinv * o48
  v0_ref[...] = v0
  q_ref[...] = (1.0 - _ALPHA) * dinv * dinv
  p_ref[...] = _ALPHA * v0


def _mlp_kernel(x_pad, W1, b1, W2, b2, deg16):
  grid = (_N_PAD // _RB,)
  sds = jax.ShapeDtypeStruct((_N_PAD, _C), jnp.float32)
  return pl.pallas_call(
      _mlp_body,
      grid=grid,
      in_specs=[
          pl.BlockSpec((_RB, _IN_C), lambda i: (i, 0)),
          pl.BlockSpec((_IN_C, _HID_C), lambda i: (0, 0)),
          pl.BlockSpec((1, _HID_C), lambda i: (0, 0)),
          pl.BlockSpec((_HID_C, _OUT_C), lambda i: (0, 0)),
          pl.BlockSpec((1, _OUT_C), lambda i: (0, 0)),
          pl.BlockSpec((_RB, _DEG_W), lambda i: (i, 0)),
      ],
      out_specs=[
          pl.BlockSpec((_RB, _C), lambda i: (i, 0)),
          pl.BlockSpec((_RB, 1), lambda i: (i, 0)),
          pl.BlockSpec((_RB, _C), lambda i: (i, 0)),
      ],
      out_shape=[sds, jax.ShapeDtypeStruct((_N_PAD, 1), jnp.float32), sds],
  )(x_pad, W1, b1, W2, b2, deg16)


def _lsm_body(v_ref, deg_ref, o_ref):
  s = jnp.sqrt(deg_ref[:, 0:1] + 1.0)
  z = v_ref[:, :_OUT_C] * s
  m = jnp.max(z, axis=1, keepdims=True)
  e = jnp.exp(z - m)
  lse = jnp.log(jnp.sum(e, axis=1, keepdims=True))
  o_ref[...] = z - m - lse


def _lsm_kernel(vK, deg16):
  grid = (_N_PAD // _RB,)
  return pl.pallas_call(
      _lsm_body,
      grid=grid,
      in_specs=[
          pl.BlockSpec((_RB, _C), lambda i: (i, 0)),
          pl.BlockSpec((_RB, _DEG_W), lambda i: (i, 0)),
      ],
      out_specs=pl.BlockSpec((_RB, _OUT_C), lambda i: (i, 0)),
      out_shape=jax.ShapeDtypeStruct((_N_PAD, _OUT_C), jnp.float32),
  )(vK, deg16)


def kernel(x, edge_index, W1, b1, W2, b2):
  src = edge_index[0].astype(jnp.int32)
  dst = edge_index[1].astype(jnp.int32)
  pad = jnp.full((_E_PAD - _E,), _N_PAD - 1, jnp.int32)
  src_e = jnp.concatenate([src, pad]).reshape(_NT, _NB, _BB)
  dst_e = jnp.concatenate([dst, pad]).reshape(_NT, _NB, _BB)
  self_ids = jnp.arange(_N_PAD, dtype=jnp.int32).reshape(_NT, _NSELF, _BB)
  dummy = jnp.full((_NT, 4, _BB), _N_PAD - 1, jnp.int32)
  srcp = jnp.concatenate([src_e, self_ids, dummy], axis=1)
  srcp = srcp.reshape(_NT, _NBE // 2 + 2, 2 * _BB)
  dstp = jnp.concatenate([dst_e, self_ids], axis=1)
  x_pad = jnp.pad(x, ((0, _N_PAD - _N), (0, 0)))

  deg16 = _deg_kernel(dst_e)
  v0, q, p = _mlp_kernel(x_pad, W1, b1.reshape(1, -1), W2,
                         b2.reshape(1, -1), deg16)
  vK = _prop_kernel(srcp, dstp, v0, q.reshape(_N_PAD), p)
  out = _lsm_kernel(vK, deg16)
  return out[:_N]


# speedup vs baseline: 1.3092x; 1.3092x over previous
import functools

import jax
import jax.numpy as jnp
from jax import lax
from jax.experimental import pallas as pl
from jax.experimental.pallas import tpu as pltpu
from jax.experimental.pallas import tpu_sc as plsc

_N = 10000
_E = 320000
_IN_C = 128
_HID_C = 64
_OUT_C = 40
_K = 10
_ALPHA = 0.1

_NT = 16
_NS = 640
_N_PAD = _NT * _NS
_C = 48
_BB = 128
_NB = 159
_EPT = _NB * _BB
_E_PAD = _NT * _EPT
_NSELF = 5
_NBE = _NB + _NSELF
_NCHUNK = _NS // _BB

_DEG_W = 16

_mesh = plsc.VectorSubcoreMesh(
    core_axis_name="c", subcore_axis_name="s", num_cores=1)
_sc_params = pltpu.CompilerParams(
    use_tc_tiling_on_sc=False, needs_layout_passes=False)


def _deg_body(dst_hbm, deg_out, dst_l, ones_b, deg_sh):
  tid = lax.axis_index("s")
  base = tid * _NS
  pltpu.sync_copy(dst_hbm.at[tid], dst_l)

  @pl.loop(0, _BB)
  def _(r):
    ones_b[r, pl.ds(0, 16)] = jnp.zeros((16,), jnp.float32)

  @pl.loop(0, _NCHUNK)
  def _(i):
    pltpu.sync_copy(ones_b, deg_sh.at[pl.ds(base + i * _BB, _BB)])

  @pl.loop(0, _BB)
  def _(r):
    ones_b[r, pl.ds(0, 16)] = jnp.ones((16,), jnp.float32)

  plsc.subcore_barrier()

  @pl.loop(0, _NB)
  def _(j):
    pltpu.sync_copy(ones_b, deg_sh.at[dst_l.at[j]], add=True)

  plsc.subcore_barrier()
  pltpu.sync_copy(deg_sh.at[pl.ds(base, _NS)], deg_out.at[pl.ds(base, _NS)])


_deg_kernel = functools.partial(
    pl.kernel,
    out_type=jax.ShapeDtypeStruct((_N_PAD, _DEG_W), jnp.float32),
    mesh=_mesh,
    compiler_params=_sc_params,
    scratch_types=[
        pltpu.VMEM((_NB, _BB), jnp.int32),
        pltpu.VMEM((_BB, _DEG_W), jnp.float32),
        pltpu.VMEM_SHARED((_N_PAD, _DEG_W), jnp.float32),
    ],
)(_deg_body)


def _prop_body(src_hbm, dst_hbm, v0_hbm, q_hbm, p_hbm, out_hbm,
               src_l, dst_l, q_l, b0, b1, vA, vB, gs0, gs1, ss0, ss1):
  tid = lax.axis_index("s")
  base = tid * _NS
  pltpu.sync_copy(src_hbm.at[tid], src_l)
  pltpu.sync_copy(dst_hbm.at[tid], dst_l)
  pltpu.sync_copy(q_hbm.at[pl.ds(base, _NS)], q_l)
  pltpu.sync_copy(v0_hbm.at[pl.ds(base, _NS)], vA.at[pl.ds(base, _NS)])

  def drain_g(buf, sem):
    pltpu.make_async_copy(vA.at[pl.ds(0, _BB)], buf, sem).wait()

  def drain_s(buf, sem):
    pltpu.make_async_copy(buf, vA.at[pl.ds(0, _BB)], sem).wait()

  for t in range(_K):
    rd, wr = (vA, vB) if t % 2 == 0 else (vB, vA)

    @pl.loop(0, _BB)
    def _(r):
      for c in range(_C // 16):
        b0[r, pl.ds(c * 16, 16)] = jnp.zeros((16,), jnp.float32)

    @pl.loop(0, _NCHUNK)
    def _(i):
      pltpu.sync_copy(b0, wr.at[pl.ds(base + i * _BB, _BB)])

    plsc.subcore_barrier()

    pltpu.async_copy(rd.at[src_l.at[0]], b0, gs0)

    @pl.loop(0, _NBE // 2)
    def _(i):
      j = 2 * i
      drain_g(b0, gs0)
      pltpu.async_copy(b0, wr.at[dst_l.at[j]], ss0, add=True)

      @pl.when(i > 0)
      def _():
        drain_s(b1, ss1)

      pltpu.async_copy(rd.at[src_l.at[j + 1]], b1, gs1)
      drain_g(b1, gs1)
      pltpu.async_copy(b1, wr.at[dst_l.at[j + 1]], ss1, add=True)
      drain_s(b0, ss0)
      pltpu.async_copy(rd.at[src_l.at[j + 2]], b0, gs0)

    drain_g(b0, gs0)
    drain_s(b1, ss1)

    plsc.subcore_barrier()

    @pl.loop(0, _NCHUNK)
    def _(i):
      sl = pl.ds(base + i * _BB, _BB)
      d0 = pltpu.async_copy(wr.at[sl], b0, gs0)
      d1 = pltpu.async_copy(p_hbm.at[sl], b1, gs1)
      d0.wait()
      d1.wait()

      @pl.loop(0, _BB)
      def _(r):
        q16 = plsc.load_gather(q_l, [jnp.full((16,), i * _BB + r, jnp.int32)])
        for c in range(_C // 16):
          cs = pl.ds(c * 16, 16)
          b0[r, cs] = q16 * b0[r, cs] + b1[r, cs]

      pltpu.sync_copy(b0, wr.at[sl])

  final = vA if _K % 2 == 0 else vB
  pltpu.sync_copy(final.at[pl.ds(base, _NS)], out_hbm.at[pl.ds(base, _NS)])


_prop_kernel = functools.partial(
    pl.kernel,
    out_type=jax.ShapeDtypeStruct((_N_PAD, _C), jnp.float32),
    mesh=_mesh,
    compiler_params=_sc_params,
    scratch_types=[
        pltpu.VMEM((_NBE + 4, _BB), jnp.int32),
        pltpu.VMEM((_NBE, _BB), jnp.int32),
        pltpu.VMEM((_NS,), jnp.float32),
        pltpu.VMEM((_BB, _C), jnp.float32),
        pltpu.VMEM((_BB, _C), jnp.float32),
        pltpu.VMEM_SHARED((_N_PAD, _C), jnp.float32),
        pltpu.VMEM_SHARED((_N_PAD, _C), jnp.float32),
        pltpu.SemaphoreType.DMA,
        pltpu.SemaphoreType.DMA,
        pltpu.SemaphoreType.DMA,
        pltpu.SemaphoreType.DMA,
    ],
)(_prop_body)


_RB = 1024


def _mlp_body(x_ref, w1_ref, b1_ref, w2_ref, b2_ref, deg_ref,
              v0_ref, q_ref, p_ref):
  h = jnp.maximum(jnp.dot(x_ref[...], w1_ref[...],
                          preferred_element_type=jnp.float32)
                  + b1_ref[...], 0.0)
  o = jnp.dot(h, w2_ref[...], preferred_element_type=jnp.float32) + b2_ref[...]
  o48 = jnp.concatenate([o, jnp.zeros((_RB, _C - _OUT_C), jnp.float32)],
                        axis=1)
  row = (pl.program_id(0) * _RB
         + lax.broadcasted_iota(jnp.int32, (_RB, 1), 0))
  deg = deg_ref[:, 0:1] + 1.0
  dinv = jnp.where(row < _N, lax.rsqrt(deg), 0.0)
  v0 = dinv * o48
  v0_ref[...] = v0
  q_ref[...] = (1.0 - _ALPHA) * dinv * dinv
  p_ref[...] = _ALPHA * v0


def _mlp_kernel(x_pad, W1, b1, W2, b2, deg16):
  grid = (_N_PAD // _RB,)
  sds = jax.ShapeDtypeStruct((_N_PAD, _C), jnp.float32)
  return pl.pallas_call(
      _mlp_body,
      grid=grid,
      in_specs=[
          pl.BlockSpec((_RB, _IN_C), lambda i: (i, 0)),
          pl.BlockSpec((_IN_C, _HID_C), lambda i: (0, 0)),
          pl.BlockSpec((1, _HID_C), lambda i: (0, 0)),
          pl.BlockSpec((_HID_C, _OUT_C), lambda i: (0, 0)),
          pl.BlockSpec((1, _OUT_C), lambda i: (0, 0)),
          pl.BlockSpec((_RB, _DEG_W), lambda i: (i, 0)),
      ],
      out_specs=[
          pl.BlockSpec((_RB, _C), lambda i: (i, 0)),
          pl.BlockSpec((_RB, 1), lambda i: (i, 0)),
          pl.BlockSpec((_RB, _C), lambda i: (i, 0)),
      ],
      out_shape=[sds, jax.ShapeDtypeStruct((_N_PAD, 1), jnp.float32), sds],
  )(x_pad, W1, b1, W2, b2, deg16)


def _lsm_body(v_ref, deg_ref, o_ref):
  s = jnp.sqrt(deg_ref[:, 0:1] + 1.0)
  z = v_ref[:, :_OUT_C] * s
  m = jnp.max(z, axis=1, keepdims=True)
  e = jnp.exp(z - m)
  lse = jnp.log(jnp.sum(e, axis=1, keepdims=True))
  o_ref[...] = z - m - lse


def _lsm_kernel(vK, deg16):
  grid = (_N_PAD // _RB,)
  return pl.pallas_call(
      _lsm_body,
      grid=grid,
      in_specs=[
          pl.BlockSpec((_RB, _C), lambda i: (i, 0)),
          pl.BlockSpec((_RB, _DEG_W), lambda i: (i, 0)),
      ],
      out_specs=pl.BlockSpec((_RB, _OUT_C), lambda i: (i, 0)),
      out_shape=jax.ShapeDtypeStruct((_N_PAD, _OUT_C), jnp.float32),
  )(vK, deg16)


def kernel(x, edge_index, W1, b1, W2, b2):
  src = edge_index[0].astype(jnp.int32)
  dst = edge_index[1].astype(jnp.int32)
  pad = jnp.full((_E_PAD - _E,), _N_PAD - 1, jnp.int32)
  src_e = jnp.concatenate([src, pad]).reshape(_NT, _NB, _BB)
  dst_e = jnp.concatenate([dst, pad]).reshape(_NT, _NB, _BB)
  self_ids = jnp.arange(_N_PAD, dtype=jnp.int32).reshape(_NT, _NSELF, _BB)
  dummy = jnp.full((_NT, 4, _BB), _N_PAD - 1, jnp.int32)
  srcp = jnp.concatenate([src_e, self_ids, dummy], axis=1)
  dstp = jnp.concatenate([dst_e, self_ids], axis=1)
  x_pad = jnp.pad(x, ((0, _N_PAD - _N), (0, 0)))

  deg16 = _deg_kernel(dst_e)
  v0, q, p = _mlp_kernel(x_pad, W1, b1.reshape(1, -1), W2,
                         b2.reshape(1, -1), deg16)
  vK = _prop_kernel(srcp, dstp, v0, q.reshape(_N_PAD), p)
  out = _lsm_kernel(vK, deg16)
  return out[:_N]
